# Initial kernel scaffold; baseline (speedup 1.0000x reference)
import jax
import jax.numpy as jnp
from jax.experimental import pallas as pl


def kernel(edge_weight, edge_index):
    raise NotImplementedError("write your pallas kernel here")


# speedup vs baseline: 211.5202x; 211.5202x over previous
import functools

import jax
import jax.numpy as jnp
from jax import lax
from jax.experimental import pallas as pl
from jax.experimental.pallas import tpu as pltpu
from jax.experimental.pallas import tpu_sc as plsc

N_NODES = 100000
N_PAD = 100352
N_EDGES = 6400000

NC = 2
NS = 16
NW = NC * NS
E_PER_W = N_EDGES // NW
C = 4000
NCHUNK = E_PER_W // C
NPW = N_PAD // NS

_mesh = plsc.VectorSubcoreMesh(core_axis_name="c", subcore_axis_name="s")


@functools.partial(
    pl.kernel,
    out_type=jax.ShapeDtypeStruct((2, NC, N_PAD), jnp.float32),
    mesh=_mesh,
    scratch_types=[
        pltpu.VMEM((C,), jnp.int32),
        pltpu.VMEM((C,), jnp.float32),
        pltpu.VMEM((NPW,), jnp.float32),
        pltpu.VMEM_SHARED((N_PAD,), jnp.float32),
        pltpu.VMEM_SHARED((N_PAD,), jnp.float32),
    ],
)
def _deg_kernel(src_hbm, dst_hbm, w_hbm, out_hbm, idx_v, w_v, z_v, acc_s, acc_d):
    cid = lax.axis_index("c")
    sid = lax.axis_index("s")
    off = sid * NPW

    def zbody(j, _):
        z_v[pl.ds(j * 16, 16)] = jnp.zeros((16,), jnp.float32)
        return 0

    lax.fori_loop(0, NPW // 16, zbody, 0)
    pltpu.sync_copy(z_v, acc_s.at[pl.ds(off, NPW)])
    pltpu.sync_copy(z_v, acc_d.at[pl.ds(off, NPW)])
    plsc.subcore_barrier()

    wid = cid * NS + sid

    def body(i, _):
        base = wid * E_PER_W + i * C
        pltpu.sync_copy(w_hbm.at[pl.ds(base, C)], w_v)
        pltpu.sync_copy(src_hbm.at[pl.ds(base, C)], idx_v)
        pltpu.sync_copy(w_v, acc_s.at[idx_v], add=True)
        pltpu.sync_copy(dst_hbm.at[pl.ds(base, C)], idx_v)
        pltpu.sync_copy(w_v, acc_d.at[idx_v], add=True)
        return 0

    lax.fori_loop(0, NCHUNK, body, 0)
    plsc.subcore_barrier()

    pltpu.sync_copy(acc_s.at[pl.ds(off, NPW)], out_hbm.at[0, cid, pl.ds(off, NPW)])
    pltpu.sync_copy(acc_d.at[pl.ds(off, NPW)], out_hbm.at[1, cid, pl.ds(off, NPW)])


def _norm_body(p_ref, o_ref):
    o_ref[...] = lax.rsqrt(p_ref[:, 0] + p_ref[:, 1])


def _norm_tc(partials):
    return pl.pallas_call(
        _norm_body,
        out_shape=jax.ShapeDtypeStruct((2, 784, 128), jnp.float32),
    )(partials.reshape(2, NC, 784, 128))


@functools.partial(
    pl.kernel,
    out_type=jax.ShapeDtypeStruct((N_EDGES,), jnp.float32),
    mesh=_mesh,
    scratch_types=[
        pltpu.VMEM((C,), jnp.int32),
        pltpu.VMEM((C,), jnp.int32),
        pltpu.VMEM((C,), jnp.float32),
        pltpu.VMEM((C,), jnp.float32),
        pltpu.VMEM((C,), jnp.float32),
        pltpu.VMEM_SHARED((N_PAD,), jnp.float32),
        pltpu.VMEM_SHARED((N_PAD,), jnp.float32),
    ],
)
def _edge_kernel(norm_hbm, src_hbm, dst_hbm, w_hbm, out_hbm,
                 si_v, di_v, w_v, a_v, b_v, ns_sh, nd_sh):
    cid = lax.axis_index("c")
    sid = lax.axis_index("s")
    off = sid * NPW

    pltpu.sync_copy(norm_hbm.at[0, pl.ds(off, NPW)], ns_sh.at[pl.ds(off, NPW)])
    pltpu.sync_copy(norm_hbm.at[1, pl.ds(off, NPW)], nd_sh.at[pl.ds(off, NPW)])
    plsc.subcore_barrier()

    wid = cid * NS + sid

    def body(i, _):
        base = wid * E_PER_W + i * C
        pltpu.sync_copy(src_hbm.at[pl.ds(base, C)], si_v)
        pltpu.sync_copy(dst_hbm.at[pl.ds(base, C)], di_v)
        pltpu.sync_copy(w_hbm.at[pl.ds(base, C)], w_v)
        pltpu.sync_copy(ns_sh.at[si_v], a_v)
        pltpu.sync_copy(nd_sh.at[di_v], b_v)

        def mbody(j, _2):
            sl = pl.ds(j * 16, 16)
            a_v[sl] = a_v[sl] * b_v[sl] * w_v[sl]
            return 0

        lax.fori_loop(0, C // 16, mbody, 0)
        pltpu.sync_copy(a_v, out_hbm.at[pl.ds(base, C)])
        return 0

    lax.fori_loop(0, NCHUNK, body, 0)


def kernel(edge_weight, edge_index):
    src = edge_index[0]
    dst = edge_index[1]
    partials = _deg_kernel(src, dst, edge_weight)
    norm = _norm_tc(partials)
    return _edge_kernel(norm.reshape(2, N_PAD), src, dst, edge_weight)


# speedup vs baseline: 386.5468x; 1.8275x over previous
import functools

import jax
import jax.numpy as jnp
from jax import lax
from jax.experimental import pallas as pl
from jax.experimental.pallas import tpu as pltpu
from jax.experimental.pallas import tpu_sc as plsc

N_NODES = 100000
N_PAD = 100352
N_EDGES = 6400000

NC = 2
NS = 16
NW = NC * NS

E1 = N_EDGES // NS
CB1 = 3200
NCH1 = E1 // CB1

E3 = N_EDGES // NW
CB3 = 1600
NCH3 = E3 // CB3

_mesh = plsc.VectorSubcoreMesh(core_axis_name="c", subcore_axis_name="s")


@functools.partial(
    pl.kernel,
    out_type=jax.ShapeDtypeStruct((NW * N_PAD,), jnp.float32),
    mesh=_mesh,
    compiler_params=pltpu.CompilerParams(needs_layout_passes=False),
    scratch_types=[
        pltpu.VMEM((N_PAD,), jnp.float32),
        pltpu.VMEM((CB1,), jnp.int32),
        pltpu.VMEM((CB1,), jnp.int32),
        pltpu.VMEM((CB1,), jnp.float32),
        pltpu.VMEM((CB1,), jnp.float32),
        pltpu.SemaphoreType.DMA,
        pltpu.SemaphoreType.DMA,
        pltpu.SemaphoreType.DMA,
        pltpu.SemaphoreType.DMA,
    ],
)
def _deg_kernel(idx2_hbm, w_hbm, out_hbm, tab, i0, i1, w0, w1, si0, si1, sw0, sw1):
    cid = lax.axis_index("c")
    sid = lax.axis_index("s")
    ibufs, wbufs = (i0, i1), (w0, w1)
    isems, wsems = (si0, si1), (sw0, sw1)

    @pl.loop(0, N_PAD // 64)
    def _zero(k):
        for u in range(4):
            tab[pl.ds(k * 64 + u * 16, 16)] = jnp.zeros((16,), jnp.float32)

    ebase = sid * E1
    ibase = cid * N_EDGES + ebase
    for b in range(2):
        off = b * CB1
        pltpu.async_copy(idx2_hbm.at[pl.ds(ibase + off, CB1)], ibufs[b], isems[b])
        pltpu.async_copy(w_hbm.at[pl.ds(ebase + off, CB1)], wbufs[b], wsems[b])

    @pl.loop(0, NCH1 - 1, step=2)
    def _main(g):
        for b in range(2):
            ch = g + b
            off = ch * CB1
            pltpu.make_async_copy(idx2_hbm.at[pl.ds(ibase + off, CB1)], ibufs[b], isems[b]).wait()
            pltpu.make_async_copy(w_hbm.at[pl.ds(ebase + off, CB1)], wbufs[b], wsems[b]).wait()

            @pl.loop(0, CB1 // 64)
            def _scat(k):
                for u in range(4):
                    sl = pl.ds(k * 64 + u * 16, 16)
                    plsc.addupdate_scatter(tab, [ibufs[b][sl]], wbufs[b][sl])

            nxt = ch + 2

            @pl.when(nxt < NCH1)
            def _():
                noff = nxt * CB1
                pltpu.async_copy(idx2_hbm.at[pl.ds(ibase + noff, CB1)], ibufs[b], isems[b])
                pltpu.async_copy(w_hbm.at[pl.ds(ebase + noff, CB1)], wbufs[b], wsems[b])

    loff = (NCH1 - 1) * CB1
    pltpu.make_async_copy(idx2_hbm.at[pl.ds(ibase + loff, CB1)], ibufs[0], isems[0]).wait()
    pltpu.make_async_copy(w_hbm.at[pl.ds(ebase + loff, CB1)], wbufs[0], wsems[0]).wait()

    @pl.loop(0, CB1 // 64)
    def _scat_tail(k):
        for u in range(4):
            sl = pl.ds(k * 64 + u * 16, 16)
            plsc.addupdate_scatter(tab, [ibufs[0][sl]], wbufs[0][sl])

    pltpu.sync_copy(tab, out_hbm.at[pl.ds((cid * NS + sid) * N_PAD, N_PAD)])


def _norm_body(p_ref, o_ref):
    n = lax.rsqrt(jnp.sum(p_ref[...], axis=1))
    u = lax.bitcast_convert_type(n.astype(jnp.bfloat16), jnp.uint16).astype(jnp.uint32)
    o_ref[...] = lax.bitcast_convert_type((u[0] << 16) | u[1], jnp.int32)


def _norm_tc(partials):
    return pl.pallas_call(
        _norm_body,
        out_shape=jax.ShapeDtypeStruct((784, 128), jnp.int32),
    )(partials.reshape(NC, NS, 784, 128))


@functools.partial(
    pl.kernel,
    out_type=jax.ShapeDtypeStruct((N_EDGES,), jnp.float32),
    mesh=_mesh,
    compiler_params=pltpu.CompilerParams(needs_layout_passes=False),
    scratch_types=[
        pltpu.VMEM((N_PAD,), jnp.int32),
        pltpu.VMEM((CB3,), jnp.int32),
        pltpu.VMEM((CB3,), jnp.int32),
        pltpu.VMEM((CB3,), jnp.int32),
        pltpu.VMEM((CB3,), jnp.int32),
        pltpu.VMEM((CB3,), jnp.float32),
        pltpu.VMEM((CB3,), jnp.float32),
        pltpu.VMEM((CB3,), jnp.float32),
        pltpu.VMEM((CB3,), jnp.float32),
        pltpu.SemaphoreType.DMA,
        pltpu.SemaphoreType.DMA,
        pltpu.SemaphoreType.DMA,
        pltpu.SemaphoreType.DMA,
        pltpu.SemaphoreType.DMA,
        pltpu.SemaphoreType.DMA,
        pltpu.SemaphoreType.DMA,
        pltpu.SemaphoreType.DMA,
    ],
)
def _edge_kernel(tab_hbm, idx2_hbm, w_hbm, out_hbm, tab,
                 s0, s1, d0, d1, w0, w1, o0, o1,
                 ss0, ss1, sd0, sd1, sw0, sw1, so0, so1):
    cid = lax.axis_index("c")
    sid = lax.axis_index("s")
    wid = cid * NS + sid
    sbufs, dbufs, wbufs, obufs = (s0, s1), (d0, d1), (w0, w1), (o0, o1)
    ssems, dsems, wsems, osems = (ss0, ss1), (sd0, sd1), (sw0, sw1), (so0, so1)

    pltpu.sync_copy(tab_hbm, tab)

    ebase = wid * E3
    for b in range(2):
        off = ebase + b * CB3
        pltpu.async_copy(idx2_hbm.at[pl.ds(off, CB3)], sbufs[b], ssems[b])
        pltpu.async_copy(idx2_hbm.at[pl.ds(N_EDGES + off, CB3)], dbufs[b], dsems[b])
        pltpu.async_copy(w_hbm.at[pl.ds(off, CB3)], wbufs[b], wsems[b])

    @pl.loop(0, NCH3 - 1, step=2)
    def _main(g):
        for b in range(2):
            ch = g + b
            off = ebase + ch * CB3
            pltpu.make_async_copy(idx2_hbm.at[pl.ds(off, CB3)], sbufs[b], ssems[b]).wait()
            pltpu.make_async_copy(idx2_hbm.at[pl.ds(N_EDGES + off, CB3)], dbufs[b], dsems[b]).wait()
            pltpu.make_async_copy(w_hbm.at[pl.ds(off, CB3)], wbufs[b], wsems[b]).wait()

            @pl.when(ch >= 2)
            def _():
                pltpu.make_async_copy(
                    obufs[b], out_hbm.at[pl.ds(ebase + (ch - 2) * CB3, CB3)], osems[b]
                ).wait()

            @pl.loop(0, CB3 // 64)
            def _mul(k):
                for u in range(4):
                    sl = pl.ds(k * 64 + u * 16, 16)
                    ws = plsc.load_gather(tab, [sbufs[b][sl]])
                    wd = plsc.load_gather(tab, [dbufs[b][sl]])
                    ns = plsc.bitcast(ws & jnp.int32(-65536), jnp.float32)
                    nd = plsc.bitcast(wd << 16, jnp.float32)
                    obufs[b][sl] = ns * nd * wbufs[b][sl]

            pltpu.async_copy(obufs[b], out_hbm.at[pl.ds(off, CB3)], osems[b])

            nxt = ch + 2

            @pl.when(nxt < NCH3)
            def _():
                noff = ebase + nxt * CB3
                pltpu.async_copy(idx2_hbm.at[pl.ds(noff, CB3)], sbufs[b], ssems[b])
                pltpu.async_copy(idx2_hbm.at[pl.ds(N_EDGES + noff, CB3)], dbufs[b], dsems[b])
                pltpu.async_copy(w_hbm.at[pl.ds(noff, CB3)], wbufs[b], wsems[b])

    loff = ebase + (NCH3 - 1) * CB3
    pltpu.make_async_copy(idx2_hbm.at[pl.ds(loff, CB3)], sbufs[0], ssems[0]).wait()
    pltpu.make_async_copy(idx2_hbm.at[pl.ds(N_EDGES + loff, CB3)], dbufs[0], dsems[0]).wait()
    pltpu.make_async_copy(w_hbm.at[pl.ds(loff, CB3)], wbufs[0], wsems[0]).wait()
    pltpu.make_async_copy(
        obufs[0], out_hbm.at[pl.ds(ebase + (NCH3 - 3) * CB3, CB3)], osems[0]
    ).wait()

    @pl.loop(0, CB3 // 64)
    def _mul_tail(k):
        for u in range(4):
            sl = pl.ds(k * 64 + u * 16, 16)
            ws = plsc.load_gather(tab, [sbufs[0][sl]])
            wd = plsc.load_gather(tab, [dbufs[0][sl]])
            ns = plsc.bitcast(ws & jnp.int32(-65536), jnp.float32)
            nd = plsc.bitcast(wd << 16, jnp.float32)
            obufs[0][sl] = ns * nd * wbufs[0][sl]

    pltpu.async_copy(obufs[0], out_hbm.at[pl.ds(loff, CB3)], osems[0])
    pltpu.make_async_copy(obufs[0], out_hbm.at[pl.ds(loff, CB3)], osems[0]).wait()
    pltpu.make_async_copy(
        obufs[1], out_hbm.at[pl.ds(ebase + (NCH3 - 2) * CB3, CB3)], osems[1]
    ).wait()


def kernel(edge_weight, edge_index):
    idx_flat = edge_index.reshape(2 * N_EDGES)
    partials = _deg_kernel(idx_flat, edge_weight)
    table = _norm_tc(partials.reshape(NC, NS, 784, 128))
    return _edge_kernel(table.reshape(N_PAD), idx_flat, edge_weight)
